# baseline (speedup 1.0000x reference)
import functools

import jax
import jax.numpy as jnp
from jax import lax
from jax.experimental import pallas as pl
from jax.experimental.pallas import tpu as pltpu
from jax.experimental.pallas import tpu_sc as plsc

N, E, D, H = 10000, 320000, 128, 128
NC, NS = 2, 16
NW = NC * NS
CH = 128
NCHUNK = 79
EPT = NCHUNK * CH
E_PAD = NW * EPT
DUMMY = 10200
NPAD = 10240
RPT = NPAD // NS
CPT = NPAD // NS


def _deg_body(dst_hbm, out_hbm, dstv, histv, bufv, resv):
    c = lax.axis_index("c")
    s = lax.axis_index("s")
    wid = c * NS + s
    pltpu.sync_copy(dst_hbm.at[pl.ds(wid * EPT, EPT)], dstv)

    def zero(i, _):
        histv[pl.ds(i * 16, 16)] = jnp.zeros((16,), jnp.float32)
        return _

    lax.fori_loop(0, NPAD // 16, zero, 0)

    ones16 = jnp.ones((16,), jnp.float32)

    def acc(i, _):
        idx = dstv[pl.ds(i * 16, 16)]
        plsc.addupdate_scatter(histv, [idx], ones16)
        return _

    lax.fori_loop(0, EPT // 16, acc, 0)

    pltpu.sync_copy(histv, bufv.at[s])
    plsc.subcore_barrier()
    pltpu.sync_copy(bufv.at[:, pl.ds(s * CPT, CPT)], resv)
    def red(j, _):
        v = resv[0, pl.ds(j * 16, 16)]
        for r in range(1, NS):
            v = v + resv[r, pl.ds(j * 16, 16)]
        resv[0, pl.ds(j * 16, 16)] = v
        return _

    lax.fori_loop(0, CPT // 16, red, 0)
    pltpu.sync_copy(resv.at[0], out_hbm.at[pl.ds(c * NPAD + s * CPT, CPT)])


@functools.cache
def _deg_kernel():
    mesh = plsc.VectorSubcoreMesh(core_axis_name="c", subcore_axis_name="s",
                                  num_cores=NC, num_subcores=NS)
    return pl.kernel(
        _deg_body,
        out_type=jax.ShapeDtypeStruct((NC * NPAD,), jnp.float32),
        mesh=mesh,
        scratch_types=[
            pltpu.VMEM((EPT,), jnp.int32),
            pltpu.VMEM((NPAD,), jnp.float32),
            pltpu.VMEM_SHARED((NS, NPAD), jnp.float32),
            pltpu.VMEM((NS, CPT), jnp.float32),
        ],
        compiler_params=pltpu.CompilerParams(needs_layout_passes=False),
    )


NB = 2
NI = 4
ROUNDS = (NCHUNK + NI - 1) // NI


def _scatter_body(u_hbm, src_hbm, dst_hbm, z_hbm, out_hbm,
                  sidx, didx, rows, acc, isems, gsem):
    c = lax.axis_index("c")
    s = lax.axis_index("s")
    base = (c * NS + s) * EPT
    pltpu.sync_copy(z_hbm, acc.at[pl.ds(s * RPT, RPT)])

    def idx_load(ch, b):
        pltpu.async_copy(src_hbm.at[pl.ds(base + ch * CH, CH)], sidx[b],
                         isems[b])
        pltpu.async_copy(dst_hbm.at[pl.ds(base + ch * CH, CH)], didx[b],
                         isems[b])

    def idx_wait(ch, b):
        pltpu.make_async_copy(src_hbm.at[pl.ds(base + ch * CH, CH)], sidx[b],
                              isems[b]).wait()
        pltpu.make_async_copy(dst_hbm.at[pl.ds(base + ch * CH, CH)], didx[b],
                              isems[b]).wait()

    def gather(ib, rb):
        pltpu.async_copy(u_hbm.at[sidx[ib]], rows[rb], gsem)

    def gather_wait(ib, rb):
        pltpu.make_async_copy(u_hbm.at[sidx[ib]], rows[rb], gsem).wait()

    for k in range(3):
        idx_load(k, k)
    idx_wait(0, 0)
    gather(0, 0)
    plsc.subcore_barrier()

    def round_(j, _):
        for k in range(NI):
            ch = NI * j + k
            @pl.when(ch < NCHUNK)
            def _a():
                gather_wait(k, k % NB)
                @pl.when(ch + 1 < NCHUNK)
                def _n():
                    idx_wait(ch + 1, (k + 1) % NI)
                    gather((k + 1) % NI, (k + 1) % NB)
                pltpu.sync_copy(rows[k % NB], acc.at[didx[k]], add=True)
                @pl.when(ch + 3 < NCHUNK)
                def _p():
                    idx_load(ch + 3, (k + 3) % NI)
        return _

    lax.fori_loop(0, ROUNDS, round_, 0)
    plsc.subcore_barrier()
    pltpu.sync_copy(acc.at[pl.ds(s * RPT, RPT)],
                    out_hbm.at[pl.ds(c * NPAD + s * RPT, RPT)])


@functools.cache
def _scatter_kernel():
    mesh = plsc.VectorSubcoreMesh(core_axis_name="c", subcore_axis_name="s",
                                  num_cores=NC, num_subcores=NS)
    return pl.kernel(
        _scatter_body,
        out_type=jax.ShapeDtypeStruct((NC * NPAD, H), jnp.float32),
        mesh=mesh,
        scratch_types=[
            [pltpu.VMEM((CH,), jnp.int32) for _ in range(NI)],
            [pltpu.VMEM((CH,), jnp.int32) for _ in range(NI)],
            [pltpu.VMEM((CH, H), jnp.float32) for _ in range(NB)],
            pltpu.VMEM_SHARED((NPAD, H), jnp.float32),
            [pltpu.SemaphoreType.DMA for _ in range(NI)],
            pltpu.SemaphoreType.DMA,
        ],
        compiler_params=pltpu.CompilerParams(needs_layout_passes=False),
    )


def _bn_relu(t, g, b):
    m = jnp.mean(t, axis=0, keepdims=True)
    v = jnp.mean((t - m) ** 2, axis=0, keepdims=True)
    return jnp.maximum((t - m) * lax.rsqrt(v + 1e-5) * g + b, 0.0)


def _tc1_body(x_ref, dinv_ref, W0_ref, b0_ref, g0_ref, be0_ref, W1_ref,
              u1_ref):
    h = jnp.dot(x_ref[...], W0_ref[...], preferred_element_type=jnp.float32)
    h = _bn_relu(h + b0_ref[...], g0_ref[...], be0_ref[...])
    u1_ref[...] = dinv_ref[...] * jnp.dot(
        h, W1_ref[...], preferred_element_type=jnp.float32)


def _tc2_body(aggp_ref, u1_ref, dinv_ref, c1b_ref, g1_ref, be1_ref, W2_ref,
              x1_ref, u2_ref):
    agg = aggp_ref[:N] + aggp_ref[NPAD:NPAD + N] + u1_ref[...]
    x1 = _bn_relu(dinv_ref[...] * agg + c1b_ref[...], g1_ref[...],
                  be1_ref[...])
    x1_ref[...] = x1
    u2_ref[...] = dinv_ref[...] * jnp.dot(
        x1, W2_ref[...], preferred_element_type=jnp.float32)


def _tc3_body(aggp_ref, u2_ref, x1_ref, dinv_ref, c2b_ref, g2_ref, be2_ref,
              W3_ref, b3_ref, W4_ref, b4_ref, y_ref):
    agg = aggp_ref[:N] + aggp_ref[NPAD:NPAD + N] + u2_ref[...]
    x2 = _bn_relu(dinv_ref[...] * agg + c2b_ref[...], g2_ref[...],
                  be2_ref[...]) + x1_ref[...]
    z = jnp.maximum(
        jnp.dot(x2, W3_ref[...], preferred_element_type=jnp.float32)
        + b3_ref[...], 0.0)
    y_ref[...] = jnp.dot(z, W4_ref[...],
                         preferred_element_type=jnp.float32) + b4_ref[...]


def _tc_call(body, out_shapes):
    return pl.pallas_call(body, out_shape=out_shapes)


def kernel(x, edge_index, W0, b0, g0, be0, W1, c1b, g1, be1, W2, c2b, g2,
           be2, W3, b3, W4, b4):
    pad = E_PAD - E
    src = jnp.concatenate([edge_index[0],
                           jnp.zeros((pad,), jnp.int32)])
    dst = jnp.concatenate([edge_index[1],
                           jnp.full((pad,), DUMMY, jnp.int32)])
    z = jnp.zeros((RPT, H), jnp.float32)

    degp = _deg_kernel()(dst)
    deg = degp[:N] + degp[NPAD:NPAD + N] + 1.0
    dinv = lax.rsqrt(deg).reshape(N, 1)

    u1 = _tc_call(_tc1_body, jax.ShapeDtypeStruct((N, H), jnp.float32))(
        x, dinv, W0, b0, g0, be0, W1)

    aggp1 = _scatter_kernel()(u1, src, dst, z)

    x1, u2 = _tc_call(_tc2_body, (
        jax.ShapeDtypeStruct((N, H), jnp.float32),
        jax.ShapeDtypeStruct((N, H), jnp.float32)))(
        aggp1, u1, dinv, c1b, g1, be1, W2)

    aggp2 = _scatter_kernel()(u2, src, dst, z)

    y = _tc_call(_tc3_body, jax.ShapeDtypeStruct((N, 1), jnp.float32))(
        aggp2, u2, x1, dinv, c2b, g2, be2, W3, b3, W4, b4)
    return y


# speedup vs baseline: 1.7743x; 1.7743x over previous
import functools

import jax
import jax.numpy as jnp
from jax import lax
from jax.experimental import pallas as pl
from jax.experimental.pallas import tpu as pltpu
from jax.experimental.pallas import tpu_sc as plsc

N, E, D, H = 10000, 320000, 128, 128
NC, NS = 2, 16
NW = NC * NS
CH = 80
NCHUNK = 125
EPT = NCHUNK * CH
E_PAD = NW * EPT
DUMMY = 10200
NPAD = 10240
RPT = NPAD // NS
CPT = NPAD // NS


def _deg_body(dst_hbm, out_hbm, dstv, histv, bufv, resv):
    c = lax.axis_index("c")
    s = lax.axis_index("s")
    wid = c * NS + s
    pltpu.sync_copy(dst_hbm.at[pl.ds(wid * EPT, EPT)], dstv)

    def zero(i, _):
        histv[pl.ds(i * 16, 16)] = jnp.zeros((16,), jnp.float32)
        return _

    lax.fori_loop(0, NPAD // 16, zero, 0)

    ones16 = jnp.ones((16,), jnp.float32)

    def acc(i, _):
        idx = dstv[pl.ds(i * 16, 16)]
        plsc.addupdate_scatter(histv, [idx], ones16)
        return _

    lax.fori_loop(0, EPT // 16, acc, 0)

    pltpu.sync_copy(histv, bufv.at[s])
    plsc.subcore_barrier()
    pltpu.sync_copy(bufv.at[:, pl.ds(s * CPT, CPT)], resv)
    def red(j, _):
        v = resv[0, pl.ds(j * 16, 16)]
        for r in range(1, NS):
            v = v + resv[r, pl.ds(j * 16, 16)]
        resv[0, pl.ds(j * 16, 16)] = v
        return _

    lax.fori_loop(0, CPT // 16, red, 0)
    pltpu.sync_copy(resv.at[0], out_hbm.at[pl.ds(c * NPAD + s * CPT, CPT)])


@functools.cache
def _deg_kernel():
    mesh = plsc.VectorSubcoreMesh(core_axis_name="c", subcore_axis_name="s",
                                  num_cores=NC, num_subcores=NS)
    return pl.kernel(
        _deg_body,
        out_type=jax.ShapeDtypeStruct((NC * NPAD,), jnp.float32),
        mesh=mesh,
        scratch_types=[
            pltpu.VMEM((EPT,), jnp.int32),
            pltpu.VMEM((NPAD,), jnp.float32),
            pltpu.VMEM_SHARED((NS, NPAD), jnp.float32),
            pltpu.VMEM((NS, CPT), jnp.float32),
        ],
        compiler_params=pltpu.CompilerParams(needs_layout_passes=False),
    )


NB = 3
NI = 8
UN = 24
ROUNDS = (NCHUNK + UN - 1) // UN


def _scatter_body(u_hbm, src_hbm, dst_hbm, z_hbm, out_hbm,
                  sidx, didx, rows, acc, isems, gsem, ssems):
    c = lax.axis_index("c")
    s = lax.axis_index("s")
    base = (c * NS + s) * EPT
    pltpu.sync_copy(z_hbm, acc.at[pl.ds(s * RPT, RPT)])

    def idx_load(ch, b):
        pltpu.async_copy(src_hbm.at[pl.ds(base + ch * CH, CH)], sidx[b],
                         isems[b])
        pltpu.async_copy(dst_hbm.at[pl.ds(base + ch * CH, CH)], didx[b],
                         isems[b])

    def idx_wait(ch, b):
        pltpu.make_async_copy(src_hbm.at[pl.ds(base + ch * CH, CH)], sidx[b],
                              isems[b]).wait()
        pltpu.make_async_copy(dst_hbm.at[pl.ds(base + ch * CH, CH)], didx[b],
                              isems[b]).wait()

    def gather(ib, rb):
        pltpu.async_copy(u_hbm.at[sidx[ib]], rows[rb], gsem)

    def gather_wait(ib, rb):
        pltpu.make_async_copy(u_hbm.at[sidx[ib]], rows[rb], gsem).wait()

    def scatter(rb, ib, sb):
        pltpu.async_copy(rows[rb], acc.at[didx[ib]], ssems[sb], add=True)

    def scatter_wait(rb, ib, sb):
        pltpu.make_async_copy(rows[rb], acc.at[didx[ib]], ssems[sb]).wait()

    for k in range(5):
        idx_load(k, k)
    idx_wait(0, 0)
    gather(0, 0)
    plsc.subcore_barrier()

    def round_(j, _):
        for k in range(UN):
            ch = UN * j + k
            @pl.when(ch < NCHUNK)
            def _a():
                gather_wait(k % NI, k % NB)
                @pl.when(ch >= 2)
                def _d():
                    scatter_wait((k - 2) % NB, (k - 2) % NI, k % 2)
                @pl.when(ch + 1 < NCHUNK)
                def _n():
                    idx_wait(ch + 1, (k + 1) % NI)
                    gather((k + 1) % NI, (k + 1) % NB)
                scatter(k % NB, k % NI, k % 2)
                @pl.when(ch + 5 < NCHUNK)
                def _p():
                    idx_load(ch + 5, (k + 5) % NI)
        return _

    lax.fori_loop(0, ROUNDS, round_, 0)
    for csc in range(NCHUNK - 2, NCHUNK):
        scatter_wait(csc % NB, csc % NI, csc % 2)
    plsc.subcore_barrier()
    pltpu.sync_copy(acc.at[pl.ds(s * RPT, RPT)],
                    out_hbm.at[pl.ds(c * NPAD + s * RPT, RPT)])


@functools.cache
def _scatter_kernel():
    mesh = plsc.VectorSubcoreMesh(core_axis_name="c", subcore_axis_name="s",
                                  num_cores=NC, num_subcores=NS)
    return pl.kernel(
        _scatter_body,
        out_type=jax.ShapeDtypeStruct((NC * NPAD, H), jnp.float32),
        mesh=mesh,
        scratch_types=[
            [pltpu.VMEM((CH,), jnp.int32) for _ in range(NI)],
            [pltpu.VMEM((CH,), jnp.int32) for _ in range(NI)],
            [pltpu.VMEM((CH, H), jnp.float32) for _ in range(NB)],
            pltpu.VMEM_SHARED((NPAD, H), jnp.float32),
            [pltpu.SemaphoreType.DMA for _ in range(NI)],
            pltpu.SemaphoreType.DMA,
            [pltpu.SemaphoreType.DMA for _ in range(2)],
        ],
        compiler_params=pltpu.CompilerParams(needs_layout_passes=False),
    )


def _bn_relu(t, g, b):
    m = jnp.mean(t, axis=0, keepdims=True)
    v = jnp.mean((t - m) ** 2, axis=0, keepdims=True)
    return jnp.maximum((t - m) * lax.rsqrt(v + 1e-5) * g + b, 0.0)


def _tc1_body(x_ref, dinv_ref, W0_ref, b0_ref, g0_ref, be0_ref, W1_ref,
              u1_ref):
    h = jnp.dot(x_ref[...], W0_ref[...], preferred_element_type=jnp.float32)
    h = _bn_relu(h + b0_ref[...], g0_ref[...], be0_ref[...])
    u1_ref[...] = dinv_ref[...] * jnp.dot(
        h, W1_ref[...], preferred_element_type=jnp.float32)


def _tc2_body(aggp_ref, u1_ref, dinv_ref, c1b_ref, g1_ref, be1_ref, W2_ref,
              x1_ref, u2_ref):
    agg = aggp_ref[:N] + aggp_ref[NPAD:NPAD + N] + u1_ref[...]
    x1 = _bn_relu(dinv_ref[...] * agg + c1b_ref[...], g1_ref[...],
                  be1_ref[...])
    x1_ref[...] = x1
    u2_ref[...] = dinv_ref[...] * jnp.dot(
        x1, W2_ref[...], preferred_element_type=jnp.float32)


def _tc3_body(aggp_ref, u2_ref, x1_ref, dinv_ref, c2b_ref, g2_ref, be2_ref,
              W3_ref, b3_ref, W4_ref, b4_ref, y_ref):
    agg = aggp_ref[:N] + aggp_ref[NPAD:NPAD + N] + u2_ref[...]
    x2 = _bn_relu(dinv_ref[...] * agg + c2b_ref[...], g2_ref[...],
                  be2_ref[...]) + x1_ref[...]
    z = jnp.maximum(
        jnp.dot(x2, W3_ref[...], preferred_element_type=jnp.float32)
        + b3_ref[...], 0.0)
    y_ref[...] = jnp.dot(z, W4_ref[...],
                         preferred_element_type=jnp.float32) + b4_ref[...]


def _tc_call(body, out_shapes):
    return pl.pallas_call(body, out_shape=out_shapes)


def kernel(x, edge_index, W0, b0, g0, be0, W1, c1b, g1, be1, W2, c2b, g2,
           be2, W3, b3, W4, b4):
    pad = E_PAD - E
    src = jnp.concatenate([edge_index[0],
                           jnp.zeros((pad,), jnp.int32)])
    dst = jnp.concatenate([edge_index[1],
                           jnp.full((pad,), DUMMY, jnp.int32)])
    z = jnp.zeros((RPT, H), jnp.float32)

    degp = _deg_kernel()(dst)
    deg = degp[:N] + degp[NPAD:NPAD + N] + 1.0
    dinv = lax.rsqrt(deg).reshape(N, 1)

    u1 = _tc_call(_tc1_body, jax.ShapeDtypeStruct((N, H), jnp.float32))(
        x, dinv, W0, b0, g0, be0, W1)

    aggp1 = _scatter_kernel()(u1, src, dst, z)

    x1, u2 = _tc_call(_tc2_body, (
        jax.ShapeDtypeStruct((N, H), jnp.float32),
        jax.ShapeDtypeStruct((N, H), jnp.float32)))(
        aggp1, u1, dinv, c1b, g1, be1, W2)

    aggp2 = _scatter_kernel()(u2, src, dst, z)

    y = _tc_call(_tc3_body, jax.ShapeDtypeStruct((N, 1), jnp.float32))(
        aggp2, u2, x1, dinv, c2b, g2, be2, W3, b3, W4, b4)
    return y


# speedup vs baseline: 2.2239x; 1.2534x over previous
import functools

import jax
import jax.numpy as jnp
from jax import lax
from jax.experimental import pallas as pl
from jax.experimental.pallas import tpu as pltpu
from jax.experimental.pallas import tpu_sc as plsc

N, E, D, H = 10000, 320000, 128, 128
NC, NS = 2, 16
NW = NC * NS
CH = 80
NCHUNK = 125
EPT = NCHUNK * CH
E_PAD = NW * EPT
DUMMY = 10200
NPAD = 10240
RPT = NPAD // NS
CPT = NPAD // NS


def _deg_body(dst_hbm, out_hbm, dstv, histv, bufv, resv):
    c = lax.axis_index("c")
    s = lax.axis_index("s")
    wid = c * NS + s
    pltpu.sync_copy(dst_hbm.at[pl.ds(wid * EPT, EPT)], dstv)

    def zero(i, _):
        histv[pl.ds(i * 16, 16)] = jnp.zeros((16,), jnp.float32)
        return _

    lax.fori_loop(0, NPAD // 16, zero, 0)

    ones16 = jnp.ones((16,), jnp.float32)

    def acc(i, _):
        idx = dstv[pl.ds(i * 16, 16)]
        plsc.addupdate_scatter(histv, [idx], ones16)
        return _

    lax.fori_loop(0, EPT // 16, acc, 0)

    pltpu.sync_copy(histv, bufv.at[s])
    plsc.subcore_barrier()
    pltpu.sync_copy(bufv.at[:, pl.ds(s * CPT, CPT)], resv)
    def red(j, _):
        v = resv[0, pl.ds(j * 16, 16)]
        for r in range(1, NS):
            v = v + resv[r, pl.ds(j * 16, 16)]
        resv[0, pl.ds(j * 16, 16)] = v
        return _

    lax.fori_loop(0, CPT // 16, red, 0)
    pltpu.sync_copy(resv.at[0], out_hbm.at[pl.ds(c * NPAD + s * CPT, CPT)])


@functools.cache
def _deg_kernel():
    mesh = plsc.VectorSubcoreMesh(core_axis_name="c", subcore_axis_name="s",
                                  num_cores=NC, num_subcores=NS)
    return pl.kernel(
        _deg_body,
        out_type=jax.ShapeDtypeStruct((NC * NPAD,), jnp.float32),
        mesh=mesh,
        scratch_types=[
            pltpu.VMEM((EPT,), jnp.int32),
            pltpu.VMEM((NPAD,), jnp.float32),
            pltpu.VMEM_SHARED((NS, NPAD), jnp.float32),
            pltpu.VMEM((NS, CPT), jnp.float32),
        ],
        compiler_params=pltpu.CompilerParams(needs_layout_passes=False),
    )


NB = 4
NI = 8
UN = 8
ROUNDS = (NCHUNK + UN - 1) // UN


def _scatter_body(u_hbm, src_hbm, dst_hbm, z_hbm, out_hbm,
                  sidx, didx, rows, acc, isems, gsems, ssems):
    c = lax.axis_index("c")
    s = lax.axis_index("s")
    base = (c * NS + s) * EPT
    pltpu.sync_copy(z_hbm, acc.at[pl.ds(s * RPT, RPT)])

    def idx_load(ch, b):
        pltpu.async_copy(src_hbm.at[pl.ds(base + ch * CH, CH)], sidx[b],
                         isems[b])
        pltpu.async_copy(dst_hbm.at[pl.ds(base + ch * CH, CH)], didx[b],
                         isems[b])

    def idx_wait(ch, b):
        pltpu.make_async_copy(src_hbm.at[pl.ds(base + ch * CH, CH)], sidx[b],
                              isems[b]).wait()
        pltpu.make_async_copy(dst_hbm.at[pl.ds(base + ch * CH, CH)], didx[b],
                              isems[b]).wait()

    def gather(ib, rb, gb):
        pltpu.async_copy(u_hbm.at[sidx[ib]], rows[rb], gsems[gb])

    def gather_wait(ib, rb, gb):
        pltpu.make_async_copy(u_hbm.at[sidx[ib]], rows[rb],
                              gsems[gb]).wait()

    def scatter(rb, ib, sb):
        pltpu.async_copy(rows[rb], acc.at[didx[ib]], ssems[sb], add=True)

    def scatter_wait(rb, ib, sb):
        pltpu.make_async_copy(rows[rb], acc.at[didx[ib]], ssems[sb]).wait()

    for k in range(5):
        idx_load(k, k)
    idx_wait(0, 0)
    gather(0, 0, 0)
    idx_wait(1, 1)
    gather(1, 1, 1)
    plsc.subcore_barrier()

    def round_(j, _):
        for k in range(UN):
            ch = UN * j + k
            @pl.when(ch < NCHUNK)
            def _a():
                gather_wait(k % NI, k % NB, k % 2)
                @pl.when(ch >= 2)
                def _d():
                    scatter_wait((k - 2) % NB, (k - 2) % NI, k % 2)
                @pl.when(ch + 2 < NCHUNK)
                def _n():
                    idx_wait(ch + 2, (k + 2) % NI)
                    gather((k + 2) % NI, (k + 2) % NB, k % 2)
                scatter(k % NB, k % NI, k % 2)
                @pl.when(ch + 5 < NCHUNK)
                def _p():
                    idx_load(ch + 5, (k + 5) % NI)
        return _

    lax.fori_loop(0, ROUNDS, round_, 0)
    for csc in range(NCHUNK - 2, NCHUNK):
        scatter_wait(csc % NB, csc % NI, csc % 2)
    plsc.subcore_barrier()
    pltpu.sync_copy(acc.at[pl.ds(s * RPT, RPT)],
                    out_hbm.at[pl.ds(c * NPAD + s * RPT, RPT)])


@functools.cache
def _scatter_kernel():
    mesh = plsc.VectorSubcoreMesh(core_axis_name="c", subcore_axis_name="s",
                                  num_cores=NC, num_subcores=NS)
    return pl.kernel(
        _scatter_body,
        out_type=jax.ShapeDtypeStruct((NC * NPAD, H), jnp.float32),
        mesh=mesh,
        scratch_types=[
            [pltpu.VMEM((CH,), jnp.int32) for _ in range(NI)],
            [pltpu.VMEM((CH,), jnp.int32) for _ in range(NI)],
            [pltpu.VMEM((CH, H), jnp.float32) for _ in range(NB)],
            pltpu.VMEM_SHARED((NPAD, H), jnp.float32),
            [pltpu.SemaphoreType.DMA for _ in range(NI)],
            [pltpu.SemaphoreType.DMA for _ in range(2)],
            [pltpu.SemaphoreType.DMA for _ in range(2)],
        ],
        compiler_params=pltpu.CompilerParams(needs_layout_passes=False),
    )


def _bn_relu(t, g, b):
    m = jnp.mean(t, axis=0, keepdims=True)
    v = jnp.mean((t - m) ** 2, axis=0, keepdims=True)
    return jnp.maximum((t - m) * lax.rsqrt(v + 1e-5) * g + b, 0.0)


def _tc1_body(x_ref, dinv_ref, W0_ref, b0_ref, g0_ref, be0_ref, W1_ref,
              u1_ref):
    h = jnp.dot(x_ref[...], W0_ref[...], preferred_element_type=jnp.float32)
    h = _bn_relu(h + b0_ref[...], g0_ref[...], be0_ref[...])
    u1_ref[...] = dinv_ref[...] * jnp.dot(
        h, W1_ref[...], preferred_element_type=jnp.float32)


def _tc2_body(aggp_ref, u1_ref, dinv_ref, c1b_ref, g1_ref, be1_ref, W2_ref,
              x1_ref, u2_ref):
    agg = aggp_ref[:N] + aggp_ref[NPAD:NPAD + N] + u1_ref[...]
    x1 = _bn_relu(dinv_ref[...] * agg + c1b_ref[...], g1_ref[...],
                  be1_ref[...])
    x1_ref[...] = x1
    u2_ref[...] = dinv_ref[...] * jnp.dot(
        x1, W2_ref[...], preferred_element_type=jnp.float32)


def _tc3_body(aggp_ref, u2_ref, x1_ref, dinv_ref, c2b_ref, g2_ref, be2_ref,
              W3_ref, b3_ref, W4_ref, b4_ref, y_ref):
    agg = aggp_ref[:N] + aggp_ref[NPAD:NPAD + N] + u2_ref[...]
    x2 = _bn_relu(dinv_ref[...] * agg + c2b_ref[...], g2_ref[...],
                  be2_ref[...]) + x1_ref[...]
    z = jnp.maximum(
        jnp.dot(x2, W3_ref[...], preferred_element_type=jnp.float32)
        + b3_ref[...], 0.0)
    y_ref[...] = jnp.dot(z, W4_ref[...],
                         preferred_element_type=jnp.float32) + b4_ref[...]


def _tc_call(body, out_shapes):
    return pl.pallas_call(body, out_shape=out_shapes)


def kernel(x, edge_index, W0, b0, g0, be0, W1, c1b, g1, be1, W2, c2b, g2,
           be2, W3, b3, W4, b4):
    pad = E_PAD - E
    src = jnp.concatenate([edge_index[0],
                           jnp.zeros((pad,), jnp.int32)])
    dst = jnp.concatenate([edge_index[1],
                           jnp.full((pad,), DUMMY, jnp.int32)])
    z = jnp.zeros((RPT, H), jnp.float32)

    degp = _deg_kernel()(dst)
    deg = degp[:N] + degp[NPAD:NPAD + N] + 1.0
    dinv = lax.rsqrt(deg).reshape(N, 1)

    u1 = _tc_call(_tc1_body, jax.ShapeDtypeStruct((N, H), jnp.float32))(
        x, dinv, W0, b0, g0, be0, W1)

    aggp1 = _scatter_kernel()(u1, src, dst, z)

    x1, u2 = _tc_call(_tc2_body, (
        jax.ShapeDtypeStruct((N, H), jnp.float32),
        jax.ShapeDtypeStruct((N, H), jnp.float32)))(
        aggp1, u1, dinv, c1b, g1, be1, W2)

    aggp2 = _scatter_kernel()(u2, src, dst, z)

    y = _tc_call(_tc3_body, jax.ShapeDtypeStruct((N, 1), jnp.float32))(
        aggp2, u2, x1, dinv, c2b, g2, be2, W3, b3, W4, b4)
    return y


# speedup vs baseline: 2.5653x; 1.1535x over previous
import functools

import jax
import jax.numpy as jnp
from jax import lax
from jax.experimental import pallas as pl
from jax.experimental.pallas import tpu as pltpu
from jax.experimental.pallas import tpu_sc as plsc

N, E, D, H = 10000, 320000, 128, 128
NC, NS = 2, 16
NW = NC * NS
CH = 80
NCHUNK = 125
EPT = NCHUNK * CH
E_PAD = NW * EPT
DUMMY = 10200
NPAD = 10240
RPT = NPAD // NS
CPT = NPAD // NS


def _deg_body(dst_hbm, out_hbm, dstv, histv, bufv, resv):
    c = lax.axis_index("c")
    s = lax.axis_index("s")
    wid = c * NS + s
    pltpu.sync_copy(dst_hbm.at[pl.ds(wid * EPT, EPT)], dstv)

    def zero(i, _):
        histv[pl.ds(i * 16, 16)] = jnp.zeros((16,), jnp.float32)
        return _

    lax.fori_loop(0, NPAD // 16, zero, 0)

    ones16 = jnp.ones((16,), jnp.float32)

    def acc(i, _):
        idx = dstv[pl.ds(i * 16, 16)]
        plsc.addupdate_scatter(histv, [idx], ones16)
        return _

    lax.fori_loop(0, EPT // 16, acc, 0)

    pltpu.sync_copy(histv, bufv.at[s])
    plsc.subcore_barrier()
    pltpu.sync_copy(bufv.at[:, pl.ds(s * CPT, CPT)], resv)
    def red(j, _):
        v = resv[0, pl.ds(j * 16, 16)]
        for r in range(1, NS):
            v = v + resv[r, pl.ds(j * 16, 16)]
        resv[0, pl.ds(j * 16, 16)] = v
        return _

    lax.fori_loop(0, CPT // 16, red, 0)
    pltpu.sync_copy(resv.at[0], out_hbm.at[pl.ds(c * NPAD + s * CPT, CPT)])


@functools.cache
def _deg_kernel():
    mesh = plsc.VectorSubcoreMesh(core_axis_name="c", subcore_axis_name="s",
                                  num_cores=NC, num_subcores=NS)
    return pl.kernel(
        _deg_body,
        out_type=jax.ShapeDtypeStruct((NC * NPAD,), jnp.float32),
        mesh=mesh,
        scratch_types=[
            pltpu.VMEM((EPT,), jnp.int32),
            pltpu.VMEM((NPAD,), jnp.float32),
            pltpu.VMEM_SHARED((NS, NPAD), jnp.float32),
            pltpu.VMEM((NS, CPT), jnp.float32),
        ],
        compiler_params=pltpu.CompilerParams(needs_layout_passes=False),
    )


NB = 4
NI = 8
NG = 3
UN = 24
ROUNDS = (NCHUNK + UN - 1) // UN


def _scatter_body(u_hbm, src_hbm, dst_hbm, z_hbm, out_hbm,
                  sidx, didx, rows, acc, isems, gsems, ssem):
    c = lax.axis_index("c")
    s = lax.axis_index("s")
    base = (c * NS + s) * EPT
    pltpu.sync_copy(z_hbm, acc.at[pl.ds(s * RPT, RPT)])

    def idx_load(ch, b):
        pltpu.async_copy(src_hbm.at[pl.ds(base + ch * CH, CH)], sidx[b],
                         isems[b])
        pltpu.async_copy(dst_hbm.at[pl.ds(base + ch * CH, CH)], didx[b],
                         isems[b])

    def idx_wait(ch, b):
        pltpu.make_async_copy(src_hbm.at[pl.ds(base + ch * CH, CH)], sidx[b],
                              isems[b]).wait()
        pltpu.make_async_copy(dst_hbm.at[pl.ds(base + ch * CH, CH)], didx[b],
                              isems[b]).wait()

    def gather(ib, rb, gb):
        pltpu.async_copy(u_hbm.at[sidx[ib]], rows[rb], gsems[gb])

    def gather_wait(ib, rb, gb):
        pltpu.make_async_copy(u_hbm.at[sidx[ib]], rows[rb],
                              gsems[gb]).wait()

    def scatter(rb, ib):
        pltpu.async_copy(rows[rb], acc.at[didx[ib]], ssem, add=True)

    def scatter_wait(rb, ib):
        pltpu.make_async_copy(rows[rb], acc.at[didx[ib]], ssem).wait()

    for k in range(5):
        idx_load(k, k)
    for k in range(NG):
        idx_wait(k, k)
        gather(k, k, k)
    plsc.subcore_barrier()

    def round_(j, _):
        for k in range(UN):
            ch = UN * j + k
            @pl.when(ch < NCHUNK)
            def _a():
                gather_wait(k % NI, k % NB, k % NG)
                @pl.when(ch >= 1)
                def _d():
                    scatter_wait((k - 1) % NB, (k - 1) % NI)
                @pl.when(ch + 3 < NCHUNK)
                def _n():
                    idx_wait(ch + 3, (k + 3) % NI)
                    gather((k + 3) % NI, (k + 3) % NB, k % NG)
                scatter(k % NB, k % NI)
                @pl.when(ch + 5 < NCHUNK)
                def _p():
                    idx_load(ch + 5, (k + 5) % NI)
        return _

    lax.fori_loop(0, ROUNDS, round_, 0)
    scatter_wait((NCHUNK - 1) % NB, (NCHUNK - 1) % NI)
    plsc.subcore_barrier()
    pltpu.sync_copy(acc.at[pl.ds(s * RPT, RPT)],
                    out_hbm.at[pl.ds(c * NPAD + s * RPT, RPT)])


@functools.cache
def _scatter_kernel():
    mesh = plsc.VectorSubcoreMesh(core_axis_name="c", subcore_axis_name="s",
                                  num_cores=NC, num_subcores=NS)
    return pl.kernel(
        _scatter_body,
        out_type=jax.ShapeDtypeStruct((NC * NPAD, H), jnp.float32),
        mesh=mesh,
        scratch_types=[
            [pltpu.VMEM((CH,), jnp.int32) for _ in range(NI)],
            [pltpu.VMEM((CH,), jnp.int32) for _ in range(NI)],
            [pltpu.VMEM((CH, H), jnp.float32) for _ in range(NB)],
            pltpu.VMEM_SHARED((NPAD, H), jnp.float32),
            [pltpu.SemaphoreType.DMA for _ in range(NI)],
            [pltpu.SemaphoreType.DMA for _ in range(NG)],
            pltpu.SemaphoreType.DMA,
        ],
        compiler_params=pltpu.CompilerParams(needs_layout_passes=False),
    )


def _bn_relu(t, g, b):
    m = jnp.mean(t, axis=0, keepdims=True)
    v = jnp.mean((t - m) ** 2, axis=0, keepdims=True)
    return jnp.maximum((t - m) * lax.rsqrt(v + 1e-5) * g + b, 0.0)


def _tc1_body(x_ref, dinv_ref, W0_ref, b0_ref, g0_ref, be0_ref, W1_ref,
              u1_ref):
    h = jnp.dot(x_ref[...], W0_ref[...], preferred_element_type=jnp.float32)
    h = _bn_relu(h + b0_ref[...], g0_ref[...], be0_ref[...])
    u1_ref[...] = dinv_ref[...] * jnp.dot(
        h, W1_ref[...], preferred_element_type=jnp.float32)


def _tc2_body(aggp_ref, u1_ref, dinv_ref, c1b_ref, g1_ref, be1_ref, W2_ref,
              x1_ref, u2_ref):
    agg = aggp_ref[:N] + aggp_ref[NPAD:NPAD + N] + u1_ref[...]
    x1 = _bn_relu(dinv_ref[...] * agg + c1b_ref[...], g1_ref[...],
                  be1_ref[...])
    x1_ref[...] = x1
    u2_ref[...] = dinv_ref[...] * jnp.dot(
        x1, W2_ref[...], preferred_element_type=jnp.float32)


def _tc3_body(aggp_ref, u2_ref, x1_ref, dinv_ref, c2b_ref, g2_ref, be2_ref,
              W3_ref, b3_ref, W4_ref, b4_ref, y_ref):
    agg = aggp_ref[:N] + aggp_ref[NPAD:NPAD + N] + u2_ref[...]
    x2 = _bn_relu(dinv_ref[...] * agg + c2b_ref[...], g2_ref[...],
                  be2_ref[...]) + x1_ref[...]
    z = jnp.maximum(
        jnp.dot(x2, W3_ref[...], preferred_element_type=jnp.float32)
        + b3_ref[...], 0.0)
    y_ref[...] = jnp.dot(z, W4_ref[...],
                         preferred_element_type=jnp.float32) + b4_ref[...]


def _tc_call(body, out_shapes):
    return pl.pallas_call(body, out_shape=out_shapes)


def kernel(x, edge_index, W0, b0, g0, be0, W1, c1b, g1, be1, W2, c2b, g2,
           be2, W3, b3, W4, b4):
    pad = E_PAD - E
    src = jnp.concatenate([edge_index[0],
                           jnp.zeros((pad,), jnp.int32)])
    dst = jnp.concatenate([edge_index[1],
                           jnp.full((pad,), DUMMY, jnp.int32)])
    z = jnp.zeros((RPT, H), jnp.float32)

    degp = _deg_kernel()(dst)
    deg = degp[:N] + degp[NPAD:NPAD + N] + 1.0
    dinv = lax.rsqrt(deg).reshape(N, 1)

    u1 = _tc_call(_tc1_body, jax.ShapeDtypeStruct((N, H), jnp.float32))(
        x, dinv, W0, b0, g0, be0, W1)

    aggp1 = _scatter_kernel()(u1, src, dst, z)

    x1, u2 = _tc_call(_tc2_body, (
        jax.ShapeDtypeStruct((N, H), jnp.float32),
        jax.ShapeDtypeStruct((N, H), jnp.float32)))(
        aggp1, u1, dinv, c1b, g1, be1, W2)

    aggp2 = _scatter_kernel()(u2, src, dst, z)

    y = _tc_call(_tc3_body, jax.ShapeDtypeStruct((N, 1), jnp.float32))(
        aggp2, u2, x1, dinv, c2b, g2, be2, W3, b3, W4, b4)
    return y
